# baseline (speedup 1.0000x reference)
import functools

import jax
import jax.numpy as jnp
from jax import lax
from jax.experimental import pallas as pl
from jax.experimental.pallas import tpu as pltpu
from jax.experimental.pallas import tpu_sc as plsc

BATCH = 16384
HIST = 200
EMBED_DIM = 32
VOCAB = 1000000
N = BATCH * HIST

NUM_CORES = 2
NUM_SUBCORES = 16
NW = NUM_CORES * NUM_SUBCORES
PER_W = N // NW
CHUNK = 1600
NCHUNK = PER_W // CHUNK
NPAIR = NCHUNK // 2
GROUPS = CHUNK // 16


def _make_gather():
    mesh = plsc.VectorSubcoreMesh(core_axis_name="c", subcore_axis_name="s")

    @functools.partial(
        pl.kernel,
        mesh=mesh,
        out_type=jax.ShapeDtypeStruct((N, 2, 16), jnp.float32),
        compiler_params=pltpu.CompilerParams(use_tc_tiling_on_sc=False),
        scratch_types=[
            pltpu.VMEM((CHUNK,), jnp.int32),
            pltpu.VMEM((CHUNK,), jnp.int32),
            pltpu.VMEM((CHUNK,), jnp.int32),
            pltpu.VMEM((CHUNK,), jnp.int32),
            pltpu.VMEM((CHUNK,), jnp.int32),
            pltpu.VMEM((CHUNK,), jnp.int32),
            pltpu.VMEM((CHUNK, 16), jnp.float32),
            pltpu.VMEM((CHUNK, 16), jnp.float32),
            pltpu.VMEM((CHUNK, 16), jnp.float32),
            pltpu.VMEM((CHUNK, 16), jnp.float32),
            pltpu.SemaphoreType.DMA,
            pltpu.SemaphoreType.DMA,
            pltpu.SemaphoreType.DMA,
            pltpu.SemaphoreType.DMA,
        ],
    )
    def gather_kernel(
        idx_hbm, table16, out16,
        idxr0, idxr1, lo0, lo1, hi0, hi1,
        rlo0, rlo1, rhi0, rhi1,
        sem_g0, sem_g1, sem_o0, sem_o1,
    ):
        wid = lax.axis_index("s") * NUM_CORES + lax.axis_index("c")
        base = wid * PER_W

        def chunk_off(g):
            return base + g * CHUNK

        def load_idx(g, idxr, lo, hi):
            pltpu.sync_copy(idx_hbm.at[pl.ds(chunk_off(g), CHUNK)], idxr)
            def lg(t, carry):
                j0 = t * 16
                v2 = idxr[pl.ds(j0, 16)] * 2
                lo[pl.ds(j0, 16)] = v2
                hi[pl.ds(j0, 16)] = v2 + 1
                return carry
            lax.fori_loop(0, GROUPS, lg, 0)

        def gather_start(lo, hi, rlo, rhi, sem):
            pltpu.async_copy(table16.at[lo], rlo, sem)
            pltpu.async_copy(table16.at[hi], rhi, sem)

        def gather_wait(lo, hi, rlo, rhi, sem):
            pltpu.make_async_copy(table16.at[lo], rlo, sem).wait()
            pltpu.make_async_copy(table16.at[hi], rhi, sem).wait()

        def out_lo(g):
            return out16.at[pl.ds(chunk_off(g), CHUNK), 0]

        def out_hi(g):
            return out16.at[pl.ds(chunk_off(g), CHUNK), 1]

        def write_start(g, rlo, rhi, sem):
            pltpu.async_copy(rlo, out_lo(g), sem)
            pltpu.async_copy(rhi, out_hi(g), sem)

        def write_wait(g, rlo, rhi, sem):
            pltpu.make_async_copy(rlo, out_lo(g), sem).wait()
            pltpu.make_async_copy(rhi, out_hi(g), sem).wait()

        load_idx(0, idxr0, lo0, hi0)
        gather_start(lo0, hi0, rlo0, rhi0, sem_g0)

        def body(p, carry):
            g0 = 2 * p
            g1 = g0 + 1
            load_idx(g1, idxr1, lo1, hi1)
            gather_wait(lo0, hi0, rlo0, rhi0, sem_g0)
            gather_start(lo1, hi1, rlo1, rhi1, sem_g1)
            write_start(g0, rlo0, rhi0, sem_o0)

            @pl.when(p < NPAIR - 1)
            def _():
                load_idx(g0 + 2, idxr0, lo0, hi0)

            write_wait(g0, rlo0, rhi0, sem_o0)

            @pl.when(p < NPAIR - 1)
            def _():
                gather_start(lo0, hi0, rlo0, rhi0, sem_g0)

            gather_wait(lo1, hi1, rlo1, rhi1, sem_g1)
            write_start(g1, rlo1, rhi1, sem_o1)
            write_wait(g1, rlo1, rhi1, sem_o1)
            return carry

        lax.fori_loop(0, NPAIR, body, 0)

    return gather_kernel


_gather = _make_gather()


def kernel(x, embed_table):
    idx = x.reshape(N)
    out = _gather(idx, embed_table.reshape(2 * VOCAB, 16))
    return out.reshape(BATCH, HIST, EMBED_DIM)


# speedup vs baseline: 7.0085x; 7.0085x over previous
import functools

import jax
import jax.numpy as jnp
from jax import lax
from jax.experimental import pallas as pl
from jax.experimental.pallas import tpu as pltpu
from jax.experimental.pallas import tpu_sc as plsc

BATCH = 16384
HIST = 200
EMBED_DIM = 32
VOCAB = 1000000
N = BATCH * HIST
ROWS_PER_LINE = 4
LINES = VOCAB // ROWS_PER_LINE

NUM_CORES = 2
NUM_SUBCORES = 16
NW = NUM_CORES * NUM_SUBCORES
PER_W = N // NW
CHUNK = 80
NCHUNK = PER_W // CHUNK
NQUAD = NCHUNK // 4
GROUPS = CHUNK // 16


def _make_gather():
    mesh = plsc.VectorSubcoreMesh(core_axis_name="c", subcore_axis_name="s")

    @functools.partial(
        pl.kernel,
        mesh=mesh,
        out_type=jax.ShapeDtypeStruct((N, EMBED_DIM), jnp.float32),
        scratch_types=[
            pltpu.VMEM((CHUNK,), jnp.int32),
            pltpu.VMEM((CHUNK,), jnp.int32),
            pltpu.VMEM((CHUNK,), jnp.int32),
            pltpu.VMEM((CHUNK,), jnp.int32),
            pltpu.VMEM((CHUNK,), jnp.int32),
            pltpu.VMEM((CHUNK,), jnp.int32),
            pltpu.VMEM((CHUNK,), jnp.int32),
            pltpu.VMEM((CHUNK,), jnp.int32),
            pltpu.VMEM((CHUNK, 128), jnp.float32),
            pltpu.VMEM((CHUNK, 128), jnp.float32),
            pltpu.VMEM((CHUNK, 128), jnp.float32),
            pltpu.VMEM((CHUNK, 128), jnp.float32),
            pltpu.VMEM((CHUNK, EMBED_DIM), jnp.float32),
            pltpu.VMEM((CHUNK, EMBED_DIM), jnp.float32),
            pltpu.SemaphoreType.DMA,
            pltpu.SemaphoreType.DMA,
            pltpu.SemaphoreType.DMA,
            pltpu.SemaphoreType.DMA,
            pltpu.SemaphoreType.DMA,
            pltpu.SemaphoreType.DMA,
        ],
    )
    def gather_kernel(
        idx_hbm, table_lines, out_hbm,
        ir0, ir1, ir2, ir3, li0, li1, li2, li3,
        ln0, ln1, ln2, ln3, ov0, ov1,
        sg0, sg1, sg2, sg3, so0, so1,
    ):
        irs = (ir0, ir1, ir2, ir3)
        lis = (li0, li1, li2, li3)
        lns = (ln0, ln1, ln2, ln3)
        ovs = (ov0, ov1)
        sgs = (sg0, sg1, sg2, sg3)
        sos = (so0, so1)

        wid = lax.axis_index("s") * NUM_CORES + lax.axis_index("c")
        base = wid * PER_W

        def chunk_off(g):
            return base + g * CHUNK

        def load_idx(g, b):
            pltpu.sync_copy(idx_hbm.at[pl.ds(chunk_off(g), CHUNK)], irs[b])
            def lg(t, carry):
                j0 = t * 16
                v = irs[b][pl.ds(j0, 16)]
                lis[b][pl.ds(j0, 16)] = lax.shift_right_logical(v, 2)
                return carry
            lax.fori_loop(0, GROUPS, lg, 0)

        def gather_start(g, b):
            pltpu.async_copy(table_lines.at[lis[b]], lns[b], sgs[b])

        def gather_wait(b):
            pltpu.make_async_copy(table_lines.at[lis[b]], lns[b], sgs[b]).wait()

        def extract(b, o):
            idxr, lines, outv = irs[b], lns[b], ovs[o]
            def grp(t, carry):
                j0 = t * 16
                qv = lax.shift_left(idxr[pl.ds(j0, 16)] & 3, 5)
                for li in range(16):
                    j = j0 + li
                    q32 = qv[li]
                    outv[j, pl.ds(0, 16)] = lines[j, pl.ds(q32, 16)]
                    outv[j, pl.ds(16, 16)] = lines[j, pl.ds(q32 + 16, 16)]
                return carry
            lax.fori_loop(0, GROUPS, grp, 0)

        def write_start(g, o):
            pltpu.async_copy(
                ovs[o], out_hbm.at[pl.ds(chunk_off(g), CHUNK)], sos[o]
            )

        def write_wait(g, o):
            pltpu.make_async_copy(
                ovs[o], out_hbm.at[pl.ds(chunk_off(g), CHUNK)], sos[o]
            ).wait()

        load_idx(0, 0)
        gather_start(0, 0)
        load_idx(1, 1)
        gather_start(1, 1)

        def body(p, carry):
            for k in range(4):
                g = 4 * p + k
                b = k
                rb = (k + 2) % 4
                o = k % 2

                if k < 2:
                    load_idx(g + 2, rb)
                    gather_start(g + 2, rb)
                else:
                    @pl.when(p < NQUAD - 1)
                    def _():
                        load_idx(g + 2, rb)
                        gather_start(g + 2, rb)

                gather_wait(b)

                if k < 2:
                    @pl.when(p > 0)
                    def _():
                        write_wait(g - 2, o)
                else:
                    write_wait(g - 2, o)

                extract(b, o)
                write_start(g, o)
            return carry

        lax.fori_loop(0, NQUAD, body, 0)

        write_wait(NCHUNK - 2, 0)
        write_wait(NCHUNK - 1, 1)

    return gather_kernel


_gather = _make_gather()


def kernel(x, embed_table):
    idx = x.reshape(N)
    out = _gather(idx, embed_table.reshape(LINES, 4 * EMBED_DIM))
    return out.reshape(BATCH, HIST, EMBED_DIM)


# speedup vs baseline: 7.6286x; 1.0885x over previous
import functools

import jax
import jax.numpy as jnp
from jax import lax
from jax.experimental import pallas as pl
from jax.experimental.pallas import tpu as pltpu
from jax.experimental.pallas import tpu_sc as plsc

BATCH = 16384
HIST = 200
EMBED_DIM = 32
VOCAB = 1000000
N = BATCH * HIST
ROWS_PER_LINE = 4
LINES = VOCAB // ROWS_PER_LINE

NUM_CORES = 2
NUM_SUBCORES = 16
NW = NUM_CORES * NUM_SUBCORES
PER_W = N // NW
CHUNK = 160
NCHUNK = PER_W // CHUNK
NTRI = (NCHUNK - 1) // 3
GROUPS = CHUNK // 16


def _make_gather():
    mesh = plsc.VectorSubcoreMesh(core_axis_name="c", subcore_axis_name="s")

    @functools.partial(
        pl.kernel,
        mesh=mesh,
        out_type=jax.ShapeDtypeStruct((N, EMBED_DIM), jnp.float32),
        scratch_types=[
            pltpu.VMEM((CHUNK,), jnp.int32),
            pltpu.VMEM((CHUNK,), jnp.int32),
            pltpu.VMEM((CHUNK,), jnp.int32),
            pltpu.VMEM((CHUNK,), jnp.int32),
            pltpu.VMEM((CHUNK,), jnp.int32),
            pltpu.VMEM((CHUNK,), jnp.int32),
            pltpu.VMEM((CHUNK, 128), jnp.float32),
            pltpu.VMEM((CHUNK, 128), jnp.float32),
            pltpu.VMEM((CHUNK, 128), jnp.float32),
            pltpu.VMEM((CHUNK, EMBED_DIM), jnp.float32),
            pltpu.SemaphoreType.DMA,
            pltpu.SemaphoreType.DMA,
            pltpu.SemaphoreType.DMA,
            pltpu.SemaphoreType.DMA,
        ],
    )
    def gather_kernel(
        idx_hbm, table_lines, out_hbm,
        ir0, ir1, ir2, li0, li1, li2,
        ln0, ln1, ln2, outv,
        sg0, sg1, sg2, so,
    ):
        irs = (ir0, ir1, ir2)
        lis = (li0, li1, li2)
        lns = (ln0, ln1, ln2)
        sgs = (sg0, sg1, sg2)

        wid = lax.axis_index("s") * NUM_CORES + lax.axis_index("c")
        base = wid * PER_W

        def chunk_off(g):
            return base + g * CHUNK

        def load_idx(g, b):
            pltpu.sync_copy(idx_hbm.at[pl.ds(chunk_off(g), CHUNK)], irs[b])
            def lg(t, carry):
                j0 = t * 16
                v = irs[b][pl.ds(j0, 16)]
                lis[b][pl.ds(j0, 16)] = lax.shift_right_logical(v, 2)
                return carry
            lax.fori_loop(0, GROUPS, lg, 0)

        def gather_start(b):
            pltpu.async_copy(table_lines.at[lis[b]], lns[b], sgs[b])

        def gather_wait(b):
            pltpu.make_async_copy(table_lines.at[lis[b]], lns[b], sgs[b]).wait()

        def extract(b):
            idxr, lines = irs[b], lns[b]
            def grp(t, carry):
                j0 = t * 16
                qv = lax.shift_left(idxr[pl.ds(j0, 16)] & 3, 5)
                for li in range(16):
                    j = j0 + li
                    q32 = qv[li]
                    outv[j, pl.ds(0, 16)] = lines[j, pl.ds(q32, 16)]
                    outv[j, pl.ds(16, 16)] = lines[j, pl.ds(q32 + 16, 16)]
                return carry
            lax.fori_loop(0, GROUPS, grp, 0)

        def write_start(g):
            pltpu.async_copy(
                outv, out_hbm.at[pl.ds(chunk_off(g), CHUNK)], so
            )

        def write_wait(g):
            pltpu.make_async_copy(
                outv, out_hbm.at[pl.ds(chunk_off(g), CHUNK)], so
            ).wait()

        load_idx(0, 0)
        gather_start(0)
        load_idx(1, 1)
        gather_start(1)

        def body(r, carry):
            for k in range(3):
                g = 3 * r + k
                b = k
                rb = (k + 2) % 3

                if k < 2:
                    load_idx(g + 2, rb)
                    gather_start(rb)
                else:
                    @pl.when(r < NTRI - 1)
                    def _():
                        load_idx(g + 2, rb)
                        gather_start(rb)

                gather_wait(b)

                if k == 0:
                    @pl.when(r > 0)
                    def _():
                        write_wait(g - 1)
                else:
                    write_wait(g - 1)

                extract(b)
                write_start(g)
            return carry

        lax.fori_loop(0, NTRI, body, 0)

        gather_wait(0)
        write_wait(NCHUNK - 2)
        extract(0)
        write_start(NCHUNK - 1)
        write_wait(NCHUNK - 1)

    return gather_kernel


_gather = _make_gather()


def kernel(x, embed_table):
    idx = x.reshape(N)
    out = _gather(idx, embed_table.reshape(LINES, 4 * EMBED_DIM))
    return out.reshape(BATCH, HIST, EMBED_DIM)


# speedup vs baseline: 7.7069x; 1.0103x over previous
import functools

import jax
import jax.numpy as jnp
from jax import lax
from jax.experimental import pallas as pl
from jax.experimental.pallas import tpu as pltpu
from jax.experimental.pallas import tpu_sc as plsc

BATCH = 16384
HIST = 200
EMBED_DIM = 32
VOCAB = 1000000
N = BATCH * HIST
ROWS_PER_LINE = 4
LINES = VOCAB // ROWS_PER_LINE

NUM_CORES = 2
NUM_SUBCORES = 16
NW = NUM_CORES * NUM_SUBCORES
PER_W = N // NW
CHUNK = 160
NCHUNK = PER_W // CHUNK
NPAIR = NCHUNK // 2
GROUPS = CHUNK // 16


def _make_gather():
    mesh = plsc.VectorSubcoreMesh(core_axis_name="c", subcore_axis_name="s")

    @functools.partial(
        pl.kernel,
        mesh=mesh,
        out_type=jax.ShapeDtypeStruct((N, EMBED_DIM), jnp.float32),
        scratch_types=[
            pltpu.VMEM((CHUNK,), jnp.int32),
            pltpu.VMEM((CHUNK,), jnp.int32),
            pltpu.VMEM((CHUNK,), jnp.int32),
            pltpu.VMEM((CHUNK,), jnp.int32),
            pltpu.VMEM((CHUNK, 128), jnp.float32),
            pltpu.VMEM((CHUNK, 128), jnp.float32),
            pltpu.VMEM((CHUNK, EMBED_DIM), jnp.float32),
            pltpu.VMEM((CHUNK, EMBED_DIM), jnp.float32),
            pltpu.SemaphoreType.DMA,
            pltpu.SemaphoreType.DMA,
            pltpu.SemaphoreType.DMA,
            pltpu.SemaphoreType.DMA,
        ],
    )
    def gather_kernel(
        idx_hbm, table_lines, out_hbm,
        idxr0, idxr1, lidx0, lidx1, lines0, lines1, outv0, outv1,
        sem_g0, sem_g1, sem_o0, sem_o1,
    ):
        wid = lax.axis_index("s") * NUM_CORES + lax.axis_index("c")
        base = wid * PER_W

        def chunk_off(g):
            return base + g * CHUNK

        def load_idx(g, idxr, lidx):
            pltpu.sync_copy(idx_hbm.at[pl.ds(chunk_off(g), CHUNK)], idxr)
            def lg(t, carry):
                j0 = t * 16
                v = idxr[pl.ds(j0, 16)]
                lidx[pl.ds(j0, 16)] = lax.shift_right_logical(v, 2)
                return carry
            lax.fori_loop(0, GROUPS, lg, 0)

        def extract(idxr, lines, outv):
            def grp(t, carry):
                j0 = t * 16
                qv = lax.shift_left(idxr[pl.ds(j0, 16)] & 3, 5)
                for li in range(16):
                    j = j0 + li
                    q32 = qv[li]
                    outv[j, pl.ds(0, 16)] = lines[j, pl.ds(q32, 16)]
                    outv[j, pl.ds(16, 16)] = lines[j, pl.ds(q32 + 16, 16)]
                return carry
            lax.fori_loop(0, GROUPS, grp, 0)

        def out_slice(g):
            return out_hbm.at[pl.ds(chunk_off(g), CHUNK)]

        load_idx(0, idxr0, lidx0)
        pltpu.async_copy(table_lines.at[lidx0], lines0, sem_g0)

        def body(p, carry):
            g0 = 2 * p
            g1 = g0 + 1
            load_idx(g1, idxr1, lidx1)
            pltpu.make_async_copy(table_lines.at[lidx0], lines0, sem_g0).wait()
            pltpu.async_copy(table_lines.at[lidx1], lines1, sem_g1)

            @pl.when(p > 0)
            def _():
                pltpu.make_async_copy(outv0, out_slice(g0), sem_o0).wait()

            extract(idxr0, lines0, outv0)
            pltpu.async_copy(outv0, out_slice(g0), sem_o0)

            @pl.when(p < NPAIR - 1)
            def _():
                load_idx(g0 + 2, idxr0, lidx0)

            pltpu.make_async_copy(table_lines.at[lidx1], lines1, sem_g1).wait()

            @pl.when(p < NPAIR - 1)
            def _():
                pltpu.async_copy(table_lines.at[lidx0], lines0, sem_g0)

            @pl.when(p > 0)
            def _():
                pltpu.make_async_copy(outv1, out_slice(g1), sem_o1).wait()

            extract(idxr1, lines1, outv1)
            pltpu.async_copy(outv1, out_slice(g1), sem_o1)
            return carry

        lax.fori_loop(0, NPAIR, body, 0)

        pltpu.make_async_copy(outv0, out_slice(NCHUNK - 2), sem_o0).wait()
        pltpu.make_async_copy(outv1, out_slice(NCHUNK - 1), sem_o1).wait()

    return gather_kernel


_gather = _make_gather()


def kernel(x, embed_table):
    idx = x.reshape(N)
    out = _gather(idx, embed_table.reshape(LINES, 128))
    return out.reshape(BATCH, HIST, EMBED_DIM)
